# Initial kernel scaffold; baseline (speedup 1.0000x reference)
import jax
import jax.numpy as jnp
from jax.experimental import pallas as pl


def kernel(z, pos, edge_index, z_emb):
    raise NotImplementedError("write your pallas kernel here")


# speedup vs baseline: 2.9051x; 2.9051x over previous
import jax
import jax.numpy as jnp
from jax import lax
from jax.experimental import pallas as pl
from jax.experimental.pallas import tpu as pltpu
from jax.experimental.pallas import tpu_sc as plsc

N_NODES = 100000
N_EDGES = 1600000
EF_DIM = 16
R_MAX = 10.0

LANES = 16
ROW = 128
NROWS = N_EDGES // ROW
NW = 32
ROWS_PER_W = -(-NROWS // NW)
G_TILES = 10
ROWS_PER_G = NROWS // G_TILES

_STEP = R_MAX / (EF_DIM - 1)
_COEFF = 0.5 / (_STEP * _STEP)
_SQC = _COEFF ** 0.5
_P = 6
_A = -(_P + 1) * (_P + 2) / 2.0
_B = float(_P * (_P + 2))
_C = -_P * (_P + 1) / 2.0

_SC_PARAMS = pltpu.CompilerParams(needs_layout_passes=False)


def _sc_dsq_body(ei, pos3, d0, d1, d2, table_v, sidx_v, didx_v, dsq_v):
    wid = lax.axis_index("c") * 16 + lax.axis_index("s")

    @pl.when(wid < 3 * G_TILES)
    def _():
        c = wid // G_TILES
        k = wid % G_TILES
        pltpu.sync_copy(pos3.at[c], table_v)
        outs = [d0, d1, d2]

        def row_body(t, carry):
            cb = k * ROWS_PER_G + t
            base = cb * ROW
            pltpu.sync_copy(ei.at[pl.ds(base, ROW)], sidx_v)
            pltpu.sync_copy(ei.at[pl.ds(N_EDGES + base, ROW)], didx_v)
            for g in range(ROW // LANES):
                si = sidx_v[pl.ds(g * LANES, LANES)]
                di = didx_v[pl.ds(g * LANES, LANES)]
                sx = plsc.load_gather(table_v, [si])
                tx = plsc.load_gather(table_v, [di])
                d = sx - tx
                dsq_v[pl.ds(g * LANES, LANES)] = d * d
            for cc in range(3):
                @pl.when(c == cc)
                def _():
                    pltpu.sync_copy(dsq_v, outs[cc].at[pl.ds(base, ROW)])
            return carry

        lax.fori_loop(0, ROWS_PER_G, row_body, 0)


def _sc_edge_body(d0, d1, d2, ef_out, sm_out,
                  d0_v, d1_v, d2_v, ef_v, sm_v):
    wid = lax.axis_index("c") * 16 + lax.axis_index("s")
    iota = lax.iota(jnp.int32, LANES)
    offs2 = iota.astype(jnp.float32) * (_STEP * _SQC)

    def row_body(t, carry):
        cb = jnp.minimum(wid * ROWS_PER_W + t, NROWS - 1)
        base = cb * ROW
        pltpu.sync_copy(d0.at[pl.ds(base, ROW)], d0_v)
        pltpu.sync_copy(d1.at[pl.ds(base, ROW)], d1_v)
        pltpu.sync_copy(d2.at[pl.ds(base, ROW)], d2_v)
        for g in range(ROW // LANES):
            el2 = (d0_v[pl.ds(g * LANES, LANES)]
                   + d1_v[pl.ds(g * LANES, LANES)]
                   + d2_v[pl.ds(g * LANES, LANES)])
            ih = plsc.bitcast(el2, jnp.int32)
            ih = 0x5F3759DF - (ih >> 1)
            y = plsc.bitcast(ih, jnp.float32)
            h = 0.5 * el2
            y = y * (1.5 - h * y * y)
            y = y * (1.5 - h * y * y)
            y = y * (1.5 - h * y * y)
            el = el2 * y
            u = el * (1.0 / R_MAX)
            u2 = u * u
            u3 = u2 * u
            u6 = u3 * u3
            inner = _A + u * (_B + _C * u)
            poly = 1.0 + u6 * inner
            sm_v[pl.ds(g * LANES, LANES)] = jnp.where(
                u < 1.0, poly, jnp.zeros_like(poly))
            els = el * _SQC
            for kk in range(LANES):
                tt = els[kk] - offs2
                ef_v[pl.ds((g * LANES + kk) * LANES, LANES)] = (
                    jnp.exp(-(tt * tt)))
        pltpu.sync_copy(ef_v, ef_out.at[pl.ds(base * EF_DIM, ROW * EF_DIM)])
        pltpu.sync_copy(sm_v, sm_out.at[pl.ds(base, ROW)])
        return carry

    lax.fori_loop(0, ROWS_PER_W, row_body, 0)


def _tc_emb_body(z_ref, zemb_ref, out_ref):
    zb = z_ref[...]
    oh = (zb == lax.broadcasted_iota(jnp.int32, (zb.shape[0], 128), 1))
    out_ref[...] = jnp.dot(oh.astype(jnp.float32), zemb_ref[...],
                           preferred_element_type=jnp.float32)


def kernel(z, pos, edge_index, z_emb):
    ei = edge_index.astype(jnp.int32).reshape(2 * N_EDGES)
    pos3 = pos.T

    mesh = plsc.VectorSubcoreMesh(core_axis_name="c", subcore_axis_name="s")

    dsq_call = pl.kernel(
        _sc_dsq_body,
        out_type=(
            jax.ShapeDtypeStruct((N_EDGES,), jnp.float32),
            jax.ShapeDtypeStruct((N_EDGES,), jnp.float32),
            jax.ShapeDtypeStruct((N_EDGES,), jnp.float32),
        ),
        mesh=mesh,
        compiler_params=_SC_PARAMS,
        scratch_types=[
            pltpu.VMEM((N_NODES,), jnp.float32),
            pltpu.VMEM((ROW,), jnp.int32),
            pltpu.VMEM((ROW,), jnp.int32),
            pltpu.VMEM((ROW,), jnp.float32),
        ],
    )
    d0, d1, d2 = dsq_call(ei, pos3)

    edge_call = pl.kernel(
        _sc_edge_body,
        out_type=(
            jax.ShapeDtypeStruct((N_EDGES * EF_DIM,), jnp.float32),
            jax.ShapeDtypeStruct((N_EDGES,), jnp.float32),
        ),
        mesh=mesh,
        compiler_params=_SC_PARAMS,
        scratch_types=[
            pltpu.VMEM((ROW,), jnp.float32),
            pltpu.VMEM((ROW,), jnp.float32),
            pltpu.VMEM((ROW,), jnp.float32),
            pltpu.VMEM((ROW * EF_DIM,), jnp.float32),
            pltpu.VMEM((ROW,), jnp.float32),
        ],
    )
    ef_flat, smooth = edge_call(d0, d1, d2)

    NB = 1024
    NPAD = -(-N_NODES // NB) * NB
    z2 = jnp.concatenate(
        [z.astype(jnp.int32), jnp.zeros((NPAD - N_NODES,), jnp.int32)]
    ).reshape(NPAD, 1)
    zemb_pad = jnp.concatenate(
        [z_emb, jnp.zeros((128 - z_emb.shape[0], 128), jnp.float32)], axis=0)
    emb1_pad = pl.pallas_call(
        _tc_emb_body,
        grid=(NPAD // NB,),
        in_specs=[
            pl.BlockSpec((NB, 1), lambda i: (i, 0)),
            pl.BlockSpec((128, 128), lambda i: (0, 0)),
        ],
        out_specs=pl.BlockSpec((NB, 128), lambda i: (i, 0)),
        out_shape=jax.ShapeDtypeStruct((NPAD, 128), jnp.float32),
    )(z2, zemb_pad)
    emb1 = emb1_pad[:N_NODES]

    return (emb1, ef_flat.reshape(N_EDGES, EF_DIM), smooth.reshape(N_EDGES, 1))


# speedup vs baseline: 8.4981x; 2.9252x over previous
import jax
import jax.numpy as jnp
from jax import lax
from jax.experimental import pallas as pl
from jax.experimental.pallas import tpu as pltpu
from jax.experimental.pallas import tpu_sc as plsc

N_NODES = 100000
N_EDGES = 1600000
EF_DIM = 16
R_MAX = 10.0

LANES = 16
NW = 32
G_TILES = 10

TN = 3200
NPADT = NW * TN
CHUNK = 2000
NGR = CHUNK // LANES
E_PER_W = N_EDGES // NW
NCH2 = E_PER_W // CHUNK
E_PER_G = N_EDGES // G_TILES
NCH1 = E_PER_G // CHUNK

_STEP = R_MAX / (EF_DIM - 1)
_COEFF = 0.5 / (_STEP * _STEP)
_SQC = _COEFF ** 0.5
_P = 6
_A = -(_P + 1) * (_P + 2) / 2.0
_B = float(_P * (_P + 2))
_C = -_P * (_P + 1) / 2.0

_D2 = 2.0 ** -0.5
_RD = [float(__import__("math").exp(-(2 * d + 1) / 2.0)) for d in range(15)]

_SC_PARAMS = pltpu.CompilerParams(needs_layout_passes=False)


def _sc_transpose_body(posf, px, py, pz, in_v, x_v, y_v, z_v):
    wid = lax.axis_index("c") * 16 + lax.axis_index("s")
    pltpu.sync_copy(posf.at[pl.ds(wid * (3 * TN), 3 * TN)], in_v)
    iota3 = lax.iota(jnp.int32, LANES) * 3
    cbufs = [x_v, y_v, z_v]

    def grp(g, carry):
        base = g * (3 * LANES)
        for c in range(3):
            v = plsc.load_gather(in_v, [iota3 + (base + c)])
            cbufs[c][pl.ds(g * LANES, LANES)] = v
        return carry

    lax.fori_loop(0, TN // LANES, grp, 0)
    for c, out in enumerate((px, py, pz)):
        pltpu.sync_copy(cbufs[c], out.at[pl.ds(wid * TN, TN)])


def _sc_dsq_body(ei, px, py, pz, d0, d1, d2,
                 table_v, si_a, si_b, di_a, di_b, dq_a, dq_b,
                 sin_a, sin_b, sout_a, sout_b):
    wid = lax.axis_index("c") * 16 + lax.axis_index("s")
    si = [si_a, si_b]
    di = [di_a, di_b]
    dq = [dq_a, dq_b]
    sin = [sin_a, sin_b]
    sout = [sout_a, sout_b]

    @pl.when(wid < 3 * G_TILES)
    def _():
        c = wid // G_TILES
        k = wid % G_TILES
        for cc, tab in enumerate((px, py, pz)):
            @pl.when(c == cc)
            def _():
                pltpu.sync_copy(tab, table_v)
        outs = [d0, d1, d2]

        def in_copies(cidx, b):
            base = k * E_PER_G + cidx * CHUNK
            return (
                pltpu.make_async_copy(ei.at[pl.ds(base, CHUNK)], si[b], sin[b]),
                pltpu.make_async_copy(
                    ei.at[pl.ds(N_EDGES + base, CHUNK)], di[b], sin[b]),
            )

        def compute(b):
            @plsc.parallel_loop(0, NGR, unroll=2)
            def _(g):
                o = g * LANES
                sv = si[b][pl.ds(o, LANES)]
                dv = di[b][pl.ds(o, LANES)]
                d = (plsc.load_gather(table_v, [sv])
                     - plsc.load_gather(table_v, [dv]))
                dq[b][pl.ds(o, LANES)] = d * d

        def out_copy(cidx, b, cc):
            base = k * E_PER_G + cidx * CHUNK
            return pltpu.make_async_copy(
                dq[b], outs[cc].at[pl.ds(base, CHUNK)], sout[b])

        def step(cidx, b):
            nxt = cidx + 1

            @pl.when(nxt < NCH1)
            def _():
                for cp in in_copies(nxt, 1 - b):
                    cp.start()

            for cp in in_copies(cidx, b):
                cp.wait()

            @pl.when(cidx >= 2)
            def _():
                for cc in range(3):
                    @pl.when(c == cc)
                    def _():
                        out_copy(cidx - 2, b, cc).wait()

            compute(b)
            for cc in range(3):
                @pl.when(c == cc)
                def _():
                    out_copy(cidx, b, cc).start()

        for cp in in_copies(0, 0):
            cp.start()

        def pair(p, carry):
            step(2 * p, 0)
            step(2 * p + 1, 1)
            return carry

        lax.fori_loop(0, NCH1 // 2, pair, 0)
        for cc in range(3):
            @pl.when(c == cc)
            def _():
                out_copy(NCH1 - 2, 0, cc).wait()
                out_copy(NCH1 - 1, 1, cc).wait()


def _sc_edge_body(d0, d1, d2, ef_out, sm_out,
                  d0a, d0b, d1a, d1b, d2a, d2b,
                  ef_a, ef_b, sm_a, sm_b,
                  sin_a, sin_b, sout_a, sout_b):
    wid = lax.axis_index("c") * 16 + lax.axis_index("s")
    dbufs = [[d0a, d1a, d2a], [d0b, d1b, d2b]]
    ef = [ef_a, ef_b]
    sm = [sm_a, sm_b]
    sin = [sin_a, sin_b]
    sout = [sout_a, sout_b]
    iota = lax.iota(jnp.int32, LANES)
    offs2 = iota.astype(jnp.float32) * (_STEP * _SQC)

    def in_copies(cidx, b):
        base = wid * E_PER_W + cidx * CHUNK
        return tuple(
            pltpu.make_async_copy(src.at[pl.ds(base, CHUNK)], dst, sin[b])
            for src, dst in zip((d0, d1, d2), dbufs[b]))

    def out_copies(cidx, b):
        base = wid * E_PER_W + cidx * CHUNK
        return (
            pltpu.make_async_copy(
                ef[b], ef_out.at[pl.ds(base * EF_DIM, CHUNK * EF_DIM)],
                sout[b]),
            pltpu.make_async_copy(sm[b], sm_out.at[pl.ds(base, CHUNK)],
                                  sout[b]),
        )

    def compute(b):
        da, db_, dc = dbufs[b]

        @plsc.parallel_loop(0, NGR, unroll=2)
        def _(g):
            o = g * LANES
            el2 = (da[pl.ds(o, LANES)] + db_[pl.ds(o, LANES)]
                   + dc[pl.ds(o, LANES)])
            ih = plsc.bitcast(el2, jnp.int32)
            ih = 0x5F3759DF - (ih >> 1)
            y = plsc.bitcast(ih, jnp.float32)
            h = 0.5 * el2
            y = y * (1.5 - (h * y) * y)
            y = y * (1.5 - (h * y) * y)
            y = y * (1.5 - (h * y) * y)
            el = el2 * y
            u = el * (1.0 / R_MAX)
            u2 = u * u
            u3 = u2 * u
            u6 = u3 * u3
            inner = _A + u * (_B + _C * u)
            poly = 1.0 + u6 * inner
            sm[b][pl.ds(o, LANES)] = jnp.where(
                u < 1.0, poly, jnp.zeros_like(poly))
            a = el * _SQC
            q = jnp.exp(el * (2.0 * _SQC * _D2))
            idxv = iota * EF_DIM + (o * EF_DIM)
            w = None
            for d in range(EF_DIM):
                if d % 4 == 0:
                    oo = d * _D2
                    w = jnp.exp((a - oo) * (oo - a))
                else:
                    w = w * (q * _RD[d - 1])
                plsc.store_scatter(ef[b], [idxv + d], w)

    def step(cidx, b):
        nxt = cidx + 1

        @pl.when(nxt < NCH2)
        def _():
            for cp in in_copies(nxt, 1 - b):
                cp.start()

        for cp in in_copies(cidx, b):
            cp.wait()

        @pl.when(cidx >= 2)
        def _():
            for cp in out_copies(cidx - 2, b):
                cp.wait()

        compute(b)
        for cp in out_copies(cidx, b):
            cp.start()

    for cp in in_copies(0, 0):
        cp.start()

    def pair(p, carry):
        step(2 * p, 0)
        step(2 * p + 1, 1)
        return carry

    lax.fori_loop(0, NCH2 // 2, pair, 0)
    step(NCH2 - 1, 0)
    for cp in out_copies(NCH2 - 3, 1):
        cp.wait()
    for cp in out_copies(NCH2 - 1, 0):
        cp.wait()


def _tc_emb_body(z_ref, zemb_ref, out_ref):
    zb = z_ref[...]
    oh = (zb == lax.broadcasted_iota(jnp.int32, (zb.shape[0], 128), 1))
    out_ref[...] = jnp.dot(oh.astype(jnp.float32), zemb_ref[...],
                           preferred_element_type=jnp.float32)


def kernel(z, pos, edge_index, z_emb):
    ei = edge_index.astype(jnp.int32).reshape(2 * N_EDGES)
    posf = jnp.concatenate(
        [pos.reshape(3 * N_NODES),
         jnp.zeros((3 * (NPADT - N_NODES),), jnp.float32)])

    mesh = plsc.VectorSubcoreMesh(core_axis_name="c", subcore_axis_name="s")

    tr_call = pl.kernel(
        _sc_transpose_body,
        out_type=(jax.ShapeDtypeStruct((NPADT,), jnp.float32),) * 3,
        mesh=mesh,
        compiler_params=_SC_PARAMS,
        scratch_types=[
            pltpu.VMEM((3 * TN,), jnp.float32),
            pltpu.VMEM((TN,), jnp.float32),
            pltpu.VMEM((TN,), jnp.float32),
            pltpu.VMEM((TN,), jnp.float32),
        ],
    )
    px, py, pz = tr_call(posf)

    dsq_call = pl.kernel(
        _sc_dsq_body,
        out_type=(
            jax.ShapeDtypeStruct((N_EDGES,), jnp.float32),
            jax.ShapeDtypeStruct((N_EDGES,), jnp.float32),
            jax.ShapeDtypeStruct((N_EDGES,), jnp.float32),
        ),
        mesh=mesh,
        compiler_params=_SC_PARAMS,
        scratch_types=(
            [pltpu.VMEM((NPADT,), jnp.float32)]
            + [pltpu.VMEM((CHUNK,), jnp.int32)] * 4
            + [pltpu.VMEM((CHUNK,), jnp.float32)] * 2
            + [pltpu.SemaphoreType.DMA] * 4
        ),
    )
    d0, d1, d2 = dsq_call(ei, px, py, pz)

    edge_call = pl.kernel(
        _sc_edge_body,
        out_type=(
            jax.ShapeDtypeStruct((N_EDGES * EF_DIM,), jnp.float32),
            jax.ShapeDtypeStruct((N_EDGES,), jnp.float32),
        ),
        mesh=mesh,
        compiler_params=_SC_PARAMS,
        scratch_types=(
            [pltpu.VMEM((CHUNK,), jnp.float32)] * 6
            + [pltpu.VMEM((CHUNK * EF_DIM,), jnp.float32)] * 2
            + [pltpu.VMEM((CHUNK,), jnp.float32)] * 2
            + [pltpu.SemaphoreType.DMA] * 4
        ),
    )
    ef_flat, smooth = edge_call(d0, d1, d2)

    NB = 1000
    z2 = z.astype(jnp.int32).reshape(N_NODES, 1)
    zemb_pad = jnp.concatenate(
        [z_emb, jnp.zeros((128 - z_emb.shape[0], 128), jnp.float32)], axis=0)
    emb1 = pl.pallas_call(
        _tc_emb_body,
        grid=(N_NODES // NB,),
        in_specs=[
            pl.BlockSpec((NB, 1), lambda i: (i, 0)),
            pl.BlockSpec((128, 128), lambda i: (0, 0)),
        ],
        out_specs=pl.BlockSpec((NB, 128), lambda i: (i, 0)),
        out_shape=jax.ShapeDtypeStruct((N_NODES, 128), jnp.float32),
    )(z2, zemb_pad)

    return (emb1, ef_flat.reshape(N_EDGES, EF_DIM), smooth.reshape(N_EDGES, 1))
